# baseline (speedup 1.0000x reference)
import functools

import jax
import jax.numpy as jnp
from jax import lax
from jax.experimental import pallas as pl
from jax.experimental.pallas import tpu as pltpu
from jax.experimental.pallas import tpu_sc as plsc

_N = 10000
_D = 128
_G = 64
_C = 10
_EPS = 1e-5

_NP = 10240
_ZROW = _N
_JROW = _N + 8
_NC = 2
_NS = 16
_NW = _NC * _NS
_EGW = 128
_RPT = _NP // _NS

def _sc_mesh():
  return plsc.VectorSubcoreMesh(core_axis_name="c", subcore_axis_name="s",
                                num_cores=_NC, num_subcores=_NS)


@functools.lru_cache(maxsize=None)
def _make_scatter(eg):
  ch = eg // 2

  @functools.partial(
      pl.kernel,
      out_type=jax.ShapeDtypeStruct((_NC, _NP, _D), jnp.float32),
      mesh=_sc_mesh(),
      scratch_types=[
          pltpu.VMEM((ch, _EGW), jnp.int32),
          pltpu.VMEM((ch, _EGW), jnp.int32),
          pltpu.VMEM((_EGW, _D), jnp.float32),
          pltpu.VMEM((_EGW, _D), jnp.float32),
          pltpu.VMEM_SHARED((_NP, _D), jnp.float32),
          pltpu.SemaphoreType.DMA,
          pltpu.SemaphoreType.DMA,
          pltpu.SemaphoreType.DMA,
      ],
  )
  def body(y_hbm, srcg_hbm, dstg_hbm, z_hbm, out_hbm,
           sidx, didx, rows_a, rows_b, acc, sem_a, sem_b, sem_z):
    c = lax.axis_index("c")
    s = lax.axis_index("s")
    wid = s * _NC + c
    zero = pltpu.make_async_copy(z_hbm.at[pl.ds(s * _RPT, _RPT)],
                                 acc.at[pl.ds(s * _RPT, _RPT)], sem_z)
    zero.start()

    def gather(j, buf, sem):
      return pltpu.make_async_copy(y_hbm.at[sidx.at[j]], buf, sem)

    for t in range(2):
      c0 = t * ch
      pltpu.sync_copy(srcg_hbm.at[wid, pl.ds(c0, ch)], sidx)
      pltpu.sync_copy(dstg_hbm.at[wid, pl.ds(c0, ch)], didx)
      if t == 0:
        zero.wait()
        plsc.subcore_barrier()
      gather(0, rows_a, sem_a).start()

      def step(u, cc):
        j0 = 2 * u
        j1 = j0 + 1
        gather(j1, rows_b, sem_b).start()
        gather(j0, rows_a, sem_a).wait()
        pltpu.sync_copy(rows_a, acc.at[didx.at[j0]], add=True)

        @pl.when(j1 + 1 < ch)
        def _():
          gather(j1 + 1, rows_a, sem_a).start()

        gather(j1, rows_b, sem_b).wait()
        pltpu.sync_copy(rows_b, acc.at[didx.at[j1]], add=True)
        return cc

      lax.fori_loop(0, ch // 2, step, 0)

    plsc.subcore_barrier()
    pltpu.sync_copy(acc.at[pl.ds(s * _RPT, _RPT)],
                    out_hbm.at[c, pl.ds(s * _RPT, _RPT)])

  return body


@functools.lru_cache(maxsize=None)
def _make_deg(eg):
  ch = eg // 2

  @functools.partial(
      pl.kernel,
      out_type=jax.ShapeDtypeStruct((_NC * _NP,), jnp.float32),
      mesh=_sc_mesh(),
      scratch_types=[
          pltpu.VMEM((ch, _EGW), jnp.int32),
          pltpu.VMEM((_NP,), jnp.float32),
          pltpu.VMEM((_NS, _RPT), jnp.float32),
          pltpu.VMEM((_RPT,), jnp.float32),
          pltpu.VMEM_SHARED((_NS, _NP), jnp.float32),
      ],
      compiler_params=pltpu.CompilerParams(needs_layout_passes=False),
  )
  def body(dstg_hbm, out_hbm, didx, hist, mbuf, res, stage):
    c = lax.axis_index("c")
    s = lax.axis_index("s")
    wid = s * _NC + c

    def z(i, carry):
      hist[pl.ds(i * 16, 16)] = jnp.zeros((16,), jnp.float32)
      return carry

    lax.fori_loop(0, _NP // 16, z, 0)

    iota = lax.iota(jnp.int32, 16)

    def shift(x, idx):
      dn = lax.GatherDimensionNumbers(offset_dims=(), collapsed_slice_dims=(0,),
                                      start_index_map=(0,))
      return lax.gather(x, idx[:, None], dn, slice_sizes=(1,),
                        mode=lax.GatherScatterMode.PROMISE_IN_BOUNDS)

    def outer(t, carry):
      c0 = pl.multiple_of(t * ch, 8)
      pltpu.sync_copy(dstg_hbm.at[wid, pl.ds(c0, ch)], didx)

      def step(j, cc):
        for l in range(_EGW // 16):
          v = didx[j, pl.ds(l * 16, 16)]
          k, _ = plsc.sort_key_val(v, v)
          prev = shift(k, jnp.maximum(iota - 1, 0))
          nxt = shift(k, jnp.minimum(iota + 1, 15))
          m_first = jnp.logical_or(iota == 0, k != prev)
          m_last = jnp.logical_or(iota == 15, k != nxt)
          seg_start = plsc.cummax(jnp.where(m_first, iota, 0))
          cnt = (iota - seg_start + 1).astype(jnp.float32)
          plsc.addupdate_scatter(hist, [k], cnt, mask=m_last)
        return cc

      lax.fori_loop(0, ch, step, 0)
      return carry

    lax.fori_loop(0, 2, outer, 0)

    pltpu.sync_copy(hist, stage.at[s])
    plsc.subcore_barrier()
    pltpu.sync_copy(stage.at[:, pl.ds(s * _RPT, _RPT)], mbuf)

    def red(i, carry):
      t = mbuf[0, pl.ds(i * 16, 16)]
      for r in range(1, _NS):
        t = t + mbuf[r, pl.ds(i * 16, 16)]
      res[pl.ds(i * 16, 16)] = t
      return carry

    lax.fori_loop(0, _RPT // 16, red, 0)
    pltpu.sync_copy(res, out_hbm.at[pl.ds(c * _NP + s * _RPT, _RPT)])

  return body


def _bnf(v):
  mu = jnp.mean(v, axis=0, keepdims=True)
  var = jnp.mean((v - mu) ** 2, axis=0, keepdims=True)
  return (v - mu) * lax.rsqrt(var + _EPS) + 1e-4


def _head1_body(x_ref, wf_ref, o_ref):
  xb = _bnf(x_ref[...])
  o_ref[...] = jnp.maximum(
      jnp.dot(xb, wf_ref[...], preferred_element_type=jnp.float32), 0.0)


def _dis(degp_ref):
  dv = degp_ref[...]
  return lax.rsqrt(dv[0][:_N] + dv[1][:_N] + 1.0)


def _head2_body(h0_ref, degp_ref, w_ref, oy_ref):
  bn = _bnf(h0_ref[...])
  y = jnp.dot(bn, w_ref[...], preferred_element_type=jnp.float32)
  oy_ref[pl.ds(0, _N), :] = y * _dis(degp_ref)
  oy_ref[pl.ds(_N, _NP - _N), :] = jnp.zeros((_NP - _N, _D), jnp.float32)


def _mid_body(accp_ref, y_ref, degp_ref, b_ref, w_ref, o_ref):
  av = accp_ref[...]
  dis = _dis(degp_ref)
  t = (av[0][:_N] + av[1][:_N] + y_ref[...][:_N]) * dis + b_ref[...]
  bn = _bnf(jnp.maximum(t, 0.0))
  y = jnp.dot(bn, w_ref[...], preferred_element_type=jnp.float32)
  o_ref[pl.ds(0, _N), :] = y * dis
  o_ref[pl.ds(_N, _NP - _N), :] = jnp.zeros((_NP - _N, _D), jnp.float32)


def _tail_body(accp_ref, y_ref, degp_ref, b_ref, batch_ref, wfc_ref, bfc_ref,
               wcls_ref, bcls_ref, o_ref):
  av = accp_ref[...]
  t = (av[0][:_N] + av[1][:_N] + y_ref[...][:_N]) * _dis(degp_ref) \
      + b_ref[...]
  h = jnp.maximum(t, 0.0)
  bt = jnp.broadcast_to(batch_ref[...], (_G, _N))
  oh = (bt == lax.broadcasted_iota(jnp.int32, (_G, _N), 0)) \
      .astype(jnp.float32)
  g = jnp.dot(oh, h, preferred_element_type=jnp.float32)
  gb = _bnf(g)
  g1 = jnp.maximum(
      jnp.dot(gb, wfc_ref[...], preferred_element_type=jnp.float32)
      + bfc_ref[...], 0.0)
  g2 = _bnf(g1)
  lo = jnp.dot(g2, wcls_ref[...], preferred_element_type=jnp.float32) \
      + bcls_ref[...]
  m = jnp.max(lo, axis=1, keepdims=True)
  lse = m + jnp.log(jnp.sum(jnp.exp(lo - m), axis=1, keepdims=True))
  o_ref[...] = lo - lse


def _tc(body, out_shape, *args):
  return pl.pallas_call(body, out_shape=out_shape)(*args)


def kernel(x, edge_index, batch, Wfeat, W1, b1, W2, b2, W3, b3,
           Wfc, bfc, Wcls, bcls):
  src, dst = edge_index[0], edge_index[1]
  e = src.shape[0]
  ept = -(-e // _NW)
  eg = -(-ept // _EGW)
  eg = -(-eg // 16) * 16
  epad = _NW * eg * _EGW

  nspare = _NP - _N

  def _grouped(a, off):
    ar = jnp.arange(epad - e, dtype=a.dtype)
    fill = (_N + (ar + off) % nspare).astype(a.dtype)
    flat = jnp.concatenate([a, fill])
    ept_r = -(-e // _NW)
    if e == _NW * ept_r and epad > e:
      real = flat[:e].reshape(_NW, ept_r)
      pads = flat[e:].reshape(_NW, -1)
      flat = jnp.concatenate([real, pads], axis=1)
    return flat.reshape(_NW, -1)

  srcg = _grouped(src, 0).reshape(_NW, eg, _EGW)
  dstg = _grouped(dst, 37).reshape(_NW, eg, _EGW)
  dstg_deg = dstg

  zeros_nd = jnp.zeros((_NP, _D), jnp.float32)
  batch_row = batch.reshape(1, _N)
  b1r, b2r, b3r = b1.reshape(1, _D), b2.reshape(1, _D), b3.reshape(1, _D)
  bfcr, bclsr = bfc.reshape(1, _D), bcls.reshape(1, _C)

  degp = _make_deg(eg)(dstg_deg).reshape(_NC, _NP, 1)
  h0 = _tc(_head1_body, jax.ShapeDtypeStruct((_N, _D), jnp.float32), x, Wfeat)
  y1 = _tc(_head2_body, jax.ShapeDtypeStruct((_NP, _D), jnp.float32),
           h0, degp, W1)

  scat = _make_scatter(eg)
  acc1 = scat(y1, srcg, dstg, zeros_nd)
  y2 = _tc(_mid_body, jax.ShapeDtypeStruct((_NP, _D), jnp.float32),
           acc1, y1, degp, b1r, W2)
  acc2 = scat(y2, srcg, dstg, zeros_nd)
  y3 = _tc(_mid_body, jax.ShapeDtypeStruct((_NP, _D), jnp.float32),
           acc2, y2, degp, b2r, W3)
  acc3 = scat(y3, srcg, dstg, zeros_nd)
  return _tc(_tail_body, jax.ShapeDtypeStruct((_G, _C), jnp.float32),
             acc3, y3, degp, b3r, batch_row, Wfc, bfcr, Wcls, bclsr)


# speedup vs baseline: 1.0683x; 1.0683x over previous
import functools

import jax
import jax.numpy as jnp
import numpy as np
from jax import lax
from jax.experimental import pallas as pl
from jax.experimental.pallas import tpu as pltpu
from jax.experimental.pallas import tpu_sc as plsc

_N = 10000
_D = 128
_G = 64
_C = 10
_EPS = 1e-5

_NP = 10240
_NC = 2
_NS = 16
_NW = _NC * _NS
_EGW = 128
_RPT = _NP // _NS

def _sc_mesh():
  return plsc.VectorSubcoreMesh(core_axis_name="c", subcore_axis_name="s",
                                num_cores=_NC, num_subcores=_NS)


@functools.lru_cache(maxsize=None)
def _make_scatter(eg):
  ch = eg // 2

  @functools.partial(
      pl.kernel,
      out_type=jax.ShapeDtypeStruct((_NC, _NP, _D), jnp.float32),
      mesh=_sc_mesh(),
      scratch_types=[
          pltpu.VMEM((ch, _EGW), jnp.int32),
          pltpu.VMEM((ch, _EGW), jnp.int32),
          pltpu.VMEM((_EGW, _D), jnp.float32),
          pltpu.VMEM((_EGW, _D), jnp.float32),
          pltpu.VMEM_SHARED((_NP, _D), jnp.float32),
          pltpu.SemaphoreType.DMA,
          pltpu.SemaphoreType.DMA,
          pltpu.SemaphoreType.DMA,
      ],
  )
  def body(y_hbm, srcg_hbm, dstg_hbm, z_hbm, out_hbm,
           sidx, didx, rows_a, rows_b, acc, sem_a, sem_b, sem_z):
    c = lax.axis_index("c")
    s = lax.axis_index("s")
    wid = s * _NC + c
    zero = pltpu.make_async_copy(z_hbm.at[pl.ds(s * _RPT, _RPT)],
                                 acc.at[pl.ds(s * _RPT, _RPT)], sem_z)
    zero.start()

    def gather(j, buf, sem):
      return pltpu.make_async_copy(y_hbm.at[sidx.at[j]], buf, sem)

    for t in range(2):
      c0 = t * ch
      pltpu.sync_copy(srcg_hbm.at[wid, pl.ds(c0, ch)], sidx)
      pltpu.sync_copy(dstg_hbm.at[wid, pl.ds(c0, ch)], didx)
      if t == 0:
        zero.wait()
        plsc.subcore_barrier()
      gather(0, rows_a, sem_a).start()

      def step(u, cc):
        j0 = 2 * u
        j1 = j0 + 1
        gather(j1, rows_b, sem_b).start()
        gather(j0, rows_a, sem_a).wait()
        pltpu.sync_copy(rows_a, acc.at[didx.at[j0]], add=True)

        @pl.when(j1 + 1 < ch)
        def _():
          gather(j1 + 1, rows_a, sem_a).start()

        gather(j1, rows_b, sem_b).wait()
        pltpu.sync_copy(rows_b, acc.at[didx.at[j1]], add=True)
        return cc

      lax.fori_loop(0, ch // 2, step, 0)

    plsc.subcore_barrier()
    pltpu.sync_copy(acc.at[pl.ds(s * _RPT, _RPT)],
                    out_hbm.at[c, pl.ds(s * _RPT, _RPT)])

  return body


@functools.lru_cache(maxsize=None)
def _make_deg(eg):
  ch = eg // 2

  @functools.partial(
      pl.kernel,
      out_type=jax.ShapeDtypeStruct((_NC * _NP,), jnp.float32),
      mesh=_sc_mesh(),
      scratch_types=[
          pltpu.VMEM((ch, _EGW), jnp.int32),
          pltpu.VMEM((_NP,), jnp.float32),
          pltpu.VMEM((_NS, _RPT), jnp.float32),
          pltpu.VMEM((_RPT,), jnp.float32),
          pltpu.VMEM_SHARED((_NS, _NP), jnp.float32),
      ],
      compiler_params=pltpu.CompilerParams(needs_layout_passes=False),
  )
  def body(dstg_hbm, out_hbm, didx, hist, mbuf, res, stage):
    c = lax.axis_index("c")
    s = lax.axis_index("s")
    wid = s * _NC + c

    def z(i, carry):
      hist[pl.ds(i * 16, 16)] = jnp.zeros((16,), jnp.float32)
      return carry

    lax.fori_loop(0, _NP // 16, z, 0)

    iota = lax.iota(jnp.int32, 16)

    def shift(x, idx):
      dn = lax.GatherDimensionNumbers(offset_dims=(), collapsed_slice_dims=(0,),
                                      start_index_map=(0,))
      return lax.gather(x, idx[:, None], dn, slice_sizes=(1,),
                        mode=lax.GatherScatterMode.PROMISE_IN_BOUNDS)

    def outer(t, carry):
      c0 = pl.multiple_of(t * ch, 8)
      pltpu.sync_copy(dstg_hbm.at[wid, pl.ds(c0, ch)], didx)

      def step(j, cc):
        for l in range(_EGW // 16):
          v = didx[j, pl.ds(l * 16, 16)]
          k, _ = plsc.sort_key_val(v, v)
          prev = shift(k, jnp.maximum(iota - 1, 0))
          nxt = shift(k, jnp.minimum(iota + 1, 15))
          m_first = jnp.logical_or(iota == 0, k != prev)
          m_last = jnp.logical_or(iota == 15, k != nxt)
          seg_start = plsc.cummax(jnp.where(m_first, iota, 0))
          cnt = (iota - seg_start + 1).astype(jnp.float32)
          plsc.addupdate_scatter(hist, [k], cnt, mask=m_last)
        return cc

      lax.fori_loop(0, ch, step, 0)
      return carry

    lax.fori_loop(0, 2, outer, 0)

    pltpu.sync_copy(hist, stage.at[s])
    plsc.subcore_barrier()
    pltpu.sync_copy(stage.at[:, pl.ds(s * _RPT, _RPT)], mbuf)

    def red(i, carry):
      t = mbuf[0, pl.ds(i * 16, 16)]
      for r in range(1, _NS):
        t = t + mbuf[r, pl.ds(i * 16, 16)]
      res[pl.ds(i * 16, 16)] = t
      return carry

    lax.fori_loop(0, _RPT // 16, red, 0)
    pltpu.sync_copy(res, out_hbm.at[pl.ds(c * _NP + s * _RPT, _RPT)])

  return body


def _bnf(v):
  mu = jnp.mean(v, axis=0, keepdims=True)
  var = jnp.mean((v - mu) ** 2, axis=0, keepdims=True)
  return (v - mu) * lax.rsqrt(var + _EPS) + 1e-4


def _head_mm_body(x_ref, wf_ref, w1_ref, o_ref):
  xb = _bnf(x_ref[...])
  h0 = jnp.maximum(
      jnp.dot(xb, wf_ref[...], preferred_element_type=jnp.float32), 0.0)
  o_ref[...] = jnp.dot(_bnf(h0), w1_ref[...],
                       preferred_element_type=jnp.float32)


def _dis(degp_ref):
  deg = degp_ref[pl.ds(0, _N)] + degp_ref[pl.ds(_NP, _N)] + 1.0
  return lax.rsqrt(deg).reshape(_N, 1)


def _head_scale_body(z_ref, degp_ref, oy_ref):
  oy_ref[pl.ds(0, _N), :] = z_ref[...] * _dis(degp_ref)
  oy_ref[pl.ds(_N, _NP - _N), :] = jnp.zeros((_NP - _N, _D), jnp.float32)


def _mid_body(accp_ref, y_ref, degp_ref, b_ref, w_ref, o_ref):
  av = accp_ref[...]
  dis = _dis(degp_ref)
  t = (av[0][:_N] + av[1][:_N] + y_ref[...][:_N]) * dis + b_ref[...]
  bn = _bnf(jnp.maximum(t, 0.0))
  y = jnp.dot(bn, w_ref[...], preferred_element_type=jnp.float32)
  o_ref[pl.ds(0, _N), :] = y * dis
  o_ref[pl.ds(_N, _NP - _N), :] = jnp.zeros((_NP - _N, _D), jnp.float32)


def _tail_body(accp_ref, y_ref, degp_ref, b_ref, batch_ref, wfc_ref, bfc_ref,
               wcls_ref, bcls_ref, o_ref):
  av = accp_ref[...]
  t = (av[0][:_N] + av[1][:_N] + y_ref[...][:_N]) * _dis(degp_ref) \
      + b_ref[...]
  h = jnp.maximum(t, 0.0)
  bt = jnp.broadcast_to(batch_ref[...], (_G, _N))
  oh = (bt == lax.broadcasted_iota(jnp.int32, (_G, _N), 0)) \
      .astype(jnp.float32)
  g = jnp.dot(oh, h, preferred_element_type=jnp.float32)
  gb = _bnf(g)
  g1 = jnp.maximum(
      jnp.dot(gb, wfc_ref[...], preferred_element_type=jnp.float32)
      + bfc_ref[...], 0.0)
  g2 = _bnf(g1)
  lo = jnp.dot(g2, wcls_ref[...], preferred_element_type=jnp.float32) \
      + bcls_ref[...]
  m = jnp.max(lo, axis=1, keepdims=True)
  lse = m + jnp.log(jnp.sum(jnp.exp(lo - m), axis=1, keepdims=True))
  o_ref[...] = lo - lse


def _tc(body, out_shape, *args):
  return pl.pallas_call(body, out_shape=out_shape)(*args)


def kernel(x, edge_index, batch, Wfeat, W1, b1, W2, b2, W3, b3,
           Wfc, bfc, Wcls, bcls):
  src, dst = edge_index[0], edge_index[1]
  e = src.shape[0]
  ept = -(-e // _NW)
  eg = -(-ept // _EGW)
  eg = -(-eg // 16) * 16
  epad = _NW * eg * _EGW

  nspare = _NP - _N

  def _grouped(a, off):
    ar = np.arange(epad - e, dtype=np.int32)
    fill = jnp.asarray(_N + (ar + off) % nspare, dtype=a.dtype)
    return jnp.concatenate([a, fill]).reshape(_NW, -1)

  srcg = _grouped(src, 0).reshape(_NW, eg, _EGW)
  dstg = _grouped(dst, 37).reshape(_NW, eg, _EGW)

  zeros_nd = jnp.zeros((_NP, _D), jnp.float32)
  batch_row = batch.reshape(1, _N)
  b1r, b2r, b3r = b1.reshape(1, _D), b2.reshape(1, _D), b3.reshape(1, _D)
  bfcr, bclsr = bfc.reshape(1, _D), bcls.reshape(1, _C)

  degp = _make_deg(eg)(dstg)
  z1 = _tc(_head_mm_body, jax.ShapeDtypeStruct((_N, _D), jnp.float32),
           x, Wfeat, W1)
  y1 = _tc(_head_scale_body, jax.ShapeDtypeStruct((_NP, _D), jnp.float32),
           z1, degp)

  scat = _make_scatter(eg)
  acc1 = scat(y1, srcg, dstg, zeros_nd)
  y2 = _tc(_mid_body, jax.ShapeDtypeStruct((_NP, _D), jnp.float32),
           acc1, y1, degp, b1r, W2)
  acc2 = scat(y2, srcg, dstg, zeros_nd)
  y3 = _tc(_mid_body, jax.ShapeDtypeStruct((_NP, _D), jnp.float32),
           acc2, y2, degp, b2r, W3)
  acc3 = scat(y3, srcg, dstg, zeros_nd)
  return _tc(_tail_body, jax.ShapeDtypeStruct((_G, _C), jnp.float32),
             acc3, y3, degp, b3r, batch_row, Wfc, bfcr, Wcls, bclsr)
